# baseline (speedup 1.0000x reference)
import jax
import jax.numpy as jnp
from jax.experimental import pallas as pl
from jax.experimental.pallas import tpu as pltpu


def _dgT(x, wT):
    return jax.lax.dot_general(
        x.astype(jnp.bfloat16), wT, (((1,), (0,)), ((), ())),
        preferred_element_type=jnp.float32)


def _lstm_elem(g, c, fd):
    i = g[:, 0 * fd:1 * fd]
    f = g[:, 1 * fd:2 * fd]
    gg = g[:, 2 * fd:3 * fd]
    o = g[:, 3 * fd:4 * fd]
    c2 = jax.nn.sigmoid(f) * c + jax.nn.sigmoid(i) * jnp.tanh(gg)
    h2 = jax.nn.sigmoid(o) * jnp.tanh(c2)
    return h2, c2


def _gather_rows(ids_ref, table_ref, dst_ref, sem, n):
    def issue(i, carry):
        pltpu.make_async_copy(
            table_ref.at[ids_ref[i]], dst_ref.at[i], sem).start()
        return carry

    jax.lax.fori_loop(0, n, issue, 0)
    pltpu.make_async_copy(
        table_ref.at[pl.ds(0, n)], dst_ref.at[pl.ds(0, n)], sem).wait()


def _bilstm_core(xflat_ref, wih_ref, whh_ref, b_ref, out_ref, hf_ref, cf_ref,
                 gx_ref, S, B):
    Dh = whh_ref.shape[1]
    FD = 4 * Dh
    gx_ref[...] = (jax.lax.dot_general(
        xflat_ref[...].astype(jnp.bfloat16), wih_ref[...],
        (((1,), (0,)), ((), ())), preferred_element_type=jnp.float32)
        + b_ref[...]).astype(gx_ref.dtype)

    def step(t, carry):
        hF, cF, hB, cB = carry
        s = S - 1 - t
        gf = gx_ref[pl.ds(t * B, B), :FD] + _dgT(hF, whh_ref[0])
        gb = gx_ref[pl.ds(s * B, B), FD:] + _dgT(hB, whh_ref[1])
        hF2, cF2 = _lstm_elem(gf, cF, Dh)
        hB2, cB2 = _lstm_elem(gb, cB, Dh)
        out_ref[pl.ds(t * B, B), :Dh] = hF2
        out_ref[pl.ds(s * B, B), Dh:] = hB2
        return (hF2, cF2, hB2, cB2)

    z = jnp.zeros((B, Dh), jnp.float32)
    hF, cF, hB, cB = jax.lax.fori_loop(0, S, step, (z, z, z, z))
    hf_ref[0] = hF
    cf_ref[0] = cF
    hf_ref[1] = hB
    cf_ref[1] = cB


def _bilstm_body(seq_ref, wih_ref, whh_ref, b_ref, out_ref, hf_ref, cf_ref,
                 gx_ref):
    NTOK = seq_ref.shape[0]
    B = hf_ref.shape[1]
    _bilstm_core(seq_ref, wih_ref, whh_ref, b_ref, out_ref, hf_ref, cf_ref,
                 gx_ref, NTOK // B, B)


def _bilstm0_body(ids_ref, emb_ref, wih_ref, whh_ref, b_ref,
                  out_ref, hf_ref, cf_ref, seq_ref, gx_ref, sem):
    NTOK = seq_ref.shape[0]
    B = hf_ref.shape[1]
    _gather_rows(ids_ref, emb_ref, seq_ref, sem, NTOK)
    _bilstm_core(seq_ref, wih_ref, whh_ref, b_ref, out_ref, hf_ref, cf_ref,
                 gx_ref, NTOK // B, B)


def _bilstm_layer(seq_flat, wih, whh, b, B):
    NTOK = seq_flat.shape[0]
    Dh = whh.shape[1]
    return pl.pallas_call(
        _bilstm_body,
        out_shape=(
            jax.ShapeDtypeStruct((NTOK, 2 * Dh), jnp.float32),
            jax.ShapeDtypeStruct((2, B, Dh), jnp.float32),
            jax.ShapeDtypeStruct((2, B, Dh), jnp.float32),
        ),
        scratch_shapes=[pltpu.VMEM((NTOK, 8 * Dh), jnp.bfloat16)],
        compiler_params=pltpu.CompilerParams(
            vmem_limit_bytes=55 * 1024 * 1024),
        name="bilstm_layer",
    )(seq_flat, wih, whh, b)


def _bilstm_layer0(ids, src_emb, wih, whh, b, B):
    NTOK = ids.shape[0]
    D = src_emb.shape[1]
    Dh = whh.shape[1]
    return pl.pallas_call(
        _bilstm0_body,
        out_shape=(
            jax.ShapeDtypeStruct((NTOK, 2 * Dh), jnp.float32),
            jax.ShapeDtypeStruct((2, B, Dh), jnp.float32),
            jax.ShapeDtypeStruct((2, B, Dh), jnp.float32),
        ),
        in_specs=[
            pl.BlockSpec(memory_space=pltpu.SMEM),
            pl.BlockSpec(memory_space=pl.ANY),
            pl.BlockSpec(memory_space=pltpu.VMEM),
            pl.BlockSpec(memory_space=pltpu.VMEM),
            pl.BlockSpec(memory_space=pltpu.VMEM),
        ],
        scratch_shapes=[
            pltpu.VMEM((NTOK, D), jnp.float32),
            pltpu.VMEM((NTOK, 8 * Dh), jnp.bfloat16),
            pltpu.SemaphoreType.DMA,
        ],
        compiler_params=pltpu.CompilerParams(
            vmem_limit_bytes=55 * 1024 * 1024),
        name="bilstm_layer0_gather",
    )(ids, src_emb, wih, whh, b)


def _dec_body(ids_ref, temb_ref, enc_ref, wa_ref, wihe_ref, w0_ref,
              b0_ref, wl_ref, b12_ref,
              h_init_ref, c_init_ref, hs_ref, emb_ref, gx0_ref, sem):
    T, B, H = hs_ref.shape
    NTOK = emb_ref.shape[0]

    _gather_rows(ids_ref, temb_ref, emb_ref, sem, NTOK)
    gx0_ref[...] = (jax.lax.dot_general(
        emb_ref[...].astype(jnp.bfloat16), wihe_ref[...],
        (((1,), (0,)), ((), ())), preferred_element_type=jnp.float32)
        + b0_ref[...])

    def step(t, carry):
        h0, c0, h1, c1, h2, c2 = carry
        q = _dgT(h2, wa_ref[...])
        enc = enc_ref[...]
        scores = jnp.sum(q[:, None, :] * enc, axis=2)
        m = jnp.max(scores, axis=1, keepdims=True)
        e = jnp.exp(scores - m)
        attn = e / jnp.sum(e, axis=1, keepdims=True)
        ctx = jnp.sum(attn[:, :, None] * enc, axis=1)

        g0 = (gx0_ref[pl.ds(t * B, B), :]
              + _dgT(jnp.concatenate([ctx, h0], axis=1), w0_ref[...]))
        h0n, c0n = _lstm_elem(g0, c0, H)
        g1 = (_dgT(jnp.concatenate([h0n, h1], axis=1), wl_ref[0])
              + b12_ref[0][None, :])
        h1n, c1n = _lstm_elem(g1, c1, H)
        g2 = (_dgT(jnp.concatenate([h1n, h2], axis=1), wl_ref[1])
              + b12_ref[1][None, :])
        h2n, c2n = _lstm_elem(g2, c2, H)
        hs_ref[t] = h2n
        return (h0n, c0n, h1n, c1n, h2n, c2n)

    init = (h_init_ref[0], c_init_ref[0], h_init_ref[1], c_init_ref[1],
            h_init_ref[2], c_init_ref[2])
    jax.lax.fori_loop(0, T, step, init)


def _decoder(ids, tgt_emb, enc_bsh, wa, wihe, w0, b0, wl, b12,
             h_init, c_init, B):
    NTOK = ids.shape[0]
    H = tgt_emb.shape[1]
    T = NTOK // B
    return pl.pallas_call(
        _dec_body,
        out_shape=jax.ShapeDtypeStruct((T, B, H), jnp.float32),
        in_specs=[
            pl.BlockSpec(memory_space=pltpu.SMEM),
            pl.BlockSpec(memory_space=pl.ANY),
        ] + [pl.BlockSpec(memory_space=pltpu.VMEM)] * 9,
        scratch_shapes=[
            pltpu.VMEM((NTOK, H), jnp.float32),
            pltpu.VMEM((NTOK, 4 * H), jnp.float32),
            pltpu.SemaphoreType.DMA,
        ],
        compiler_params=pltpu.CompilerParams(
            vmem_limit_bytes=55 * 1024 * 1024),
        name="decoder_recurrence",
    )(ids, tgt_emb, enc_bsh, wa, wihe, w0, b0, wl, b12, h_init, c_init)


def _proj_body(x_ref, w_ref, b_ref, o_ref):
    o_ref[...] = (jax.lax.dot_general(
        x_ref[...].astype(jnp.bfloat16), w_ref[...],
        (((1,), (1,)), ((), ())), preferred_element_type=jnp.float32)
        + b_ref[...])


def _projection(x, w, b):
    R, H = x.shape
    V = w.shape[0]
    BV = 1280
    return pl.pallas_call(
        _proj_body,
        out_shape=jax.ShapeDtypeStruct((R, V), jnp.float32),
        grid=(V // BV,),
        in_specs=[
            pl.BlockSpec((R, H), lambda v: (0, 0)),
            pl.BlockSpec((BV, H), lambda v: (v, 0)),
            pl.BlockSpec((1, BV), lambda v: (0, v)),
        ],
        out_specs=pl.BlockSpec((R, BV), lambda v: (0, v)),
        compiler_params=pltpu.CompilerParams(
            dimension_semantics=("parallel",),
            vmem_limit_bytes=55 * 1024 * 1024),
        name="out_projection",
    )(x, w, b)


def kernel(x, y, src_emb, tgt_emb, W_ih_e0, W_hh_e0, b_e0, W_ih_e12,
           W_hh_e12, b_e12, W_ih_d0, W_hh_d0, b_d0, W_ih_d12, W_hh_d12,
           b_d12, W_a, W_out, b_out):
    B, S = x.shape
    T = y.shape[1]
    H = tgt_emb.shape[1]
    VT = W_out.shape[0]
    bf = jnp.bfloat16

    def enc_weights(wih, whh, b):
        wihT = jnp.concatenate([wih[0].T, wih[1].T], axis=1).astype(bf)
        whhT = jnp.stack([whh[0].T, whh[1].T]).astype(bf)
        bcat = jnp.concatenate([b[0], b[1]]).reshape(1, -1)
        return wihT, whhT, bcat

    src_ids = x.T.reshape(-1)
    hs, cs = [], []
    wihT, whhT, bcat = enc_weights(W_ih_e0, W_hh_e0, b_e0)
    seq, hf, cf = _bilstm_layer0(src_ids, src_emb, wihT, whhT, bcat, B)
    hs.append(hf); cs.append(cf)
    for l in range(2):
        wihT, whhT, bcat = enc_weights(W_ih_e12[l], W_hh_e12[l], b_e12[l])
        seq, hf, cf = _bilstm_layer(seq, wihT, whhT, bcat, B)
        hs.append(hf); cs.append(cf)

    enc_bsh = seq.reshape(S, B, H).transpose(1, 0, 2)
    h_init = jnp.stack([jnp.concatenate([h[0], h[1]], -1) for h in hs])
    c_init = jnp.stack([jnp.concatenate([c[0], c[1]], -1) for c in cs])

    tgt_ids = y[:, :-1].T.reshape(-1)
    w0 = jnp.concatenate([W_ih_d0[:, H:].T, W_hh_d0.T], axis=0)
    wl = jnp.stack([
        jnp.concatenate([W_ih_d12[0].T, W_hh_d12[0].T], axis=0),
        jnp.concatenate([W_ih_d12[1].T, W_hh_d12[1].T], axis=0)])
    hs_top = _decoder(
        tgt_ids, tgt_emb, enc_bsh, W_a.astype(bf),
        W_ih_d0[:, :H].T.astype(bf), w0.astype(bf), b_d0.reshape(1, -1),
        wl.astype(bf), b_d12, h_init, c_init, B)

    rows = hs_top.transpose(1, 0, 2).reshape(B * (T - 1), H)
    logits = _projection(rows, W_out.astype(bf), b_out.reshape(1, VT))
    return logits.reshape(B, T - 1, VT)


# speedup vs baseline: 1.0261x; 1.0261x over previous
import jax
import jax.numpy as jnp
from jax.experimental import pallas as pl
from jax.experimental.pallas import tpu as pltpu


def _dgT(x, wT):
    return jax.lax.dot_general(
        x.astype(jnp.bfloat16), wT, (((1,), (0,)), ((), ())),
        preferred_element_type=jnp.float32)


def _lstm_elem(g, c, fd):
    i = g[:, 0 * fd:1 * fd]
    f = g[:, 1 * fd:2 * fd]
    gg = g[:, 2 * fd:3 * fd]
    o = g[:, 3 * fd:4 * fd]
    c2 = jax.nn.sigmoid(f) * c + jax.nn.sigmoid(i) * jnp.tanh(gg)
    h2 = jax.nn.sigmoid(o) * jnp.tanh(c2)
    return h2, c2


def _gather_rows(ids_ref, table_ref, dst_ref, sem, n):
    def issue(i, carry):
        pltpu.make_async_copy(
            table_ref.at[ids_ref[i]], dst_ref.at[i], sem).start()
        return carry

    jax.lax.fori_loop(0, n, issue, 0)
    pltpu.make_async_copy(
        table_ref.at[pl.ds(0, n)], dst_ref.at[pl.ds(0, n)], sem).wait()


def _bilstm_core(xflat_ref, wih_ref, whh_ref, b_ref, out_ref, hf_ref, cf_ref,
                 gx_ref, S, B):
    Dh = whh_ref.shape[1]
    FD = 4 * Dh
    gx_ref[...] = (jax.lax.dot_general(
        xflat_ref[...].astype(jnp.bfloat16), wih_ref[...],
        (((1,), (0,)), ((), ())), preferred_element_type=jnp.float32)
        + b_ref[...]).astype(gx_ref.dtype)

    def step(t, carry):
        hF, cF, hB, cB = carry
        s = S - 1 - t
        gf = gx_ref[pl.ds(t * B, B), :FD] + _dgT(hF, whh_ref[0])
        gb = gx_ref[pl.ds(s * B, B), FD:] + _dgT(hB, whh_ref[1])
        hF2, cF2 = _lstm_elem(gf, cF, Dh)
        hB2, cB2 = _lstm_elem(gb, cB, Dh)
        out_ref[pl.ds(t * B, B), :Dh] = hF2.astype(out_ref.dtype)
        out_ref[pl.ds(s * B, B), Dh:] = hB2.astype(out_ref.dtype)
        return (hF2, cF2, hB2, cB2)

    z = jnp.zeros((B, Dh), jnp.float32)
    hF, cF, hB, cB = jax.lax.fori_loop(0, S, step, (z, z, z, z))
    hf_ref[0] = hF
    cf_ref[0] = cF
    hf_ref[1] = hB
    cf_ref[1] = cB


def _encoder_body(ids_ref, emb_ref, wih0_ref, whh0_ref, b0_ref,
                  wih1_ref, whh1_ref, b1_ref, wih2_ref, whh2_ref, b2_ref,
                  out_ref, hf_ref, cf_ref, seq0_ref, seqA_ref, seqB_ref,
                  gx_ref, sem):
    NTOK = seq0_ref.shape[0]
    B = hf_ref.shape[2]
    S = NTOK // B
    _gather_rows(ids_ref, emb_ref, seq0_ref, sem, NTOK)
    _bilstm_core(seq0_ref, wih0_ref, whh0_ref, b0_ref, seqA_ref,
                 hf_ref.at[0], cf_ref.at[0], gx_ref, S, B)
    _bilstm_core(seqA_ref, wih1_ref, whh1_ref, b1_ref, seqB_ref,
                 hf_ref.at[1], cf_ref.at[1], gx_ref, S, B)
    _bilstm_core(seqB_ref, wih2_ref, whh2_ref, b2_ref, out_ref,
                 hf_ref.at[2], cf_ref.at[2], gx_ref, S, B)


def _encoder(ids, src_emb, w0, w1, w2, B):
    NTOK = ids.shape[0]
    D = src_emb.shape[1]
    Dh = w0[1].shape[1]
    return pl.pallas_call(
        _encoder_body,
        out_shape=(
            jax.ShapeDtypeStruct((NTOK, 2 * Dh), jnp.float32),
            jax.ShapeDtypeStruct((3, 2, B, Dh), jnp.float32),
            jax.ShapeDtypeStruct((3, 2, B, Dh), jnp.float32),
        ),
        in_specs=[
            pl.BlockSpec(memory_space=pltpu.SMEM),
            pl.BlockSpec(memory_space=pl.ANY),
        ] + [pl.BlockSpec(memory_space=pltpu.VMEM)] * 9,
        scratch_shapes=[
            pltpu.VMEM((NTOK, D), jnp.float32),
            pltpu.VMEM((NTOK, 2 * Dh), jnp.bfloat16),
            pltpu.VMEM((NTOK, 2 * Dh), jnp.bfloat16),
            pltpu.VMEM((NTOK, 8 * Dh), jnp.bfloat16),
            pltpu.SemaphoreType.DMA,
        ],
        compiler_params=pltpu.CompilerParams(
            vmem_limit_bytes=55 * 1024 * 1024),
        name="encoder3_bilstm",
    )(ids, src_emb, w0[0], w0[1], w0[2], w1[0], w1[1], w1[2],
      w2[0], w2[1], w2[2])


def _dec_body(ids_ref, temb_ref, enc_ref, wa_ref, wihe_ref, wihc_ref,
              whh0_ref, b0_ref, wih12_ref, whh12_ref, b12_ref,
              h_init_ref, c_init_ref, hs_ref, emb_ref, gx0_ref, sem):
    T, B, H = hs_ref.shape
    NTOK = emb_ref.shape[0]

    _gather_rows(ids_ref, temb_ref, emb_ref, sem, NTOK)
    gx0_ref[...] = (jax.lax.dot_general(
        emb_ref[...].astype(jnp.bfloat16), wihe_ref[...],
        (((1,), (0,)), ((), ())), preferred_element_type=jnp.float32)
        + b0_ref[...])

    def step(t, carry):
        h0, c0, h1, c1, h2, c2 = carry
        q = _dgT(h2, wa_ref[...])
        enc = enc_ref[...]
        scores = jnp.sum(q[:, None, :] * enc, axis=2)
        m = jnp.max(scores, axis=1, keepdims=True)
        e = jnp.exp(scores - m)
        attn = e / jnp.sum(e, axis=1, keepdims=True)
        ctx = jnp.sum(attn[:, :, None] * enc, axis=1)

        g0 = (gx0_ref[pl.ds(t * B, B), :] + _dgT(ctx, wihc_ref[...])
              + _dgT(h0, whh0_ref[...]))
        h0n, c0n = _lstm_elem(g0, c0, H)
        g1 = (_dgT(h0n, wih12_ref[0]) + _dgT(h1, whh12_ref[0])
              + b12_ref[0][None, :])
        h1n, c1n = _lstm_elem(g1, c1, H)
        g2 = (_dgT(h1n, wih12_ref[1]) + _dgT(h2, whh12_ref[1])
              + b12_ref[1][None, :])
        h2n, c2n = _lstm_elem(g2, c2, H)
        hs_ref[t] = h2n
        return (h0n, c0n, h1n, c1n, h2n, c2n)

    init = (h_init_ref[0], c_init_ref[0], h_init_ref[1], c_init_ref[1],
            h_init_ref[2], c_init_ref[2])
    jax.lax.fori_loop(0, T, step, init)


def _decoder(ids, tgt_emb, enc_bsh, wa, wihe, wihc, whh0, b0, wih12,
             whh12, b12, h_init, c_init, B):
    NTOK = ids.shape[0]
    H = tgt_emb.shape[1]
    T = NTOK // B
    return pl.pallas_call(
        _dec_body,
        out_shape=jax.ShapeDtypeStruct((T, B, H), jnp.float32),
        in_specs=[
            pl.BlockSpec(memory_space=pltpu.SMEM),
            pl.BlockSpec(memory_space=pl.ANY),
        ] + [pl.BlockSpec(memory_space=pltpu.VMEM)] * 11,
        scratch_shapes=[
            pltpu.VMEM((NTOK, H), jnp.float32),
            pltpu.VMEM((NTOK, 4 * H), jnp.float32),
            pltpu.SemaphoreType.DMA,
        ],
        compiler_params=pltpu.CompilerParams(
            vmem_limit_bytes=55 * 1024 * 1024),
        name="decoder_recurrence",
    )(ids, tgt_emb, enc_bsh, wa, wihe, wihc, whh0, b0, wih12, whh12, b12,
      h_init, c_init)


def _proj_body(x_ref, w_ref, b_ref, o_ref):
    o_ref[...] = (jax.lax.dot_general(
        x_ref[...].astype(jnp.bfloat16), w_ref[...],
        (((1,), (1,)), ((), ())), preferred_element_type=jnp.float32)
        + b_ref[...])


def _projection(x, w, b):
    R, H = x.shape
    V = w.shape[0]
    BV = 1280
    return pl.pallas_call(
        _proj_body,
        out_shape=jax.ShapeDtypeStruct((R, V), jnp.float32),
        grid=(V // BV,),
        in_specs=[
            pl.BlockSpec((R, H), lambda v: (0, 0)),
            pl.BlockSpec((BV, H), lambda v: (v, 0)),
            pl.BlockSpec((1, BV), lambda v: (0, v)),
        ],
        out_specs=pl.BlockSpec((R, BV), lambda v: (0, v)),
        compiler_params=pltpu.CompilerParams(
            dimension_semantics=("parallel",),
            vmem_limit_bytes=55 * 1024 * 1024),
        name="out_projection",
    )(x, w, b)


def kernel(x, y, src_emb, tgt_emb, W_ih_e0, W_hh_e0, b_e0, W_ih_e12,
           W_hh_e12, b_e12, W_ih_d0, W_hh_d0, b_d0, W_ih_d12, W_hh_d12,
           b_d12, W_a, W_out, b_out):
    B, S = x.shape
    T = y.shape[1]
    H = tgt_emb.shape[1]
    VT = W_out.shape[0]
    bf = jnp.bfloat16

    def enc_weights(wih, whh, b):
        wihT = jnp.concatenate([wih[0].T, wih[1].T], axis=1).astype(bf)
        whhT = jnp.stack([whh[0].T, whh[1].T]).astype(bf)
        bcat = jnp.concatenate([b[0], b[1]]).reshape(1, -1)
        return wihT, whhT, bcat

    src_ids = x.T.reshape(-1)
    seq, hf, cf = _encoder(
        src_ids, src_emb,
        enc_weights(W_ih_e0, W_hh_e0, b_e0),
        enc_weights(W_ih_e12[0], W_hh_e12[0], b_e12[0]),
        enc_weights(W_ih_e12[1], W_hh_e12[1], b_e12[1]), B)

    enc_bsh = seq.reshape(S, B, H).transpose(1, 0, 2)
    h_init = jnp.stack(
        [jnp.concatenate([hf[l, 0], hf[l, 1]], -1) for l in range(3)])
    c_init = jnp.stack(
        [jnp.concatenate([cf[l, 0], cf[l, 1]], -1) for l in range(3)])

    tgt_ids = y[:, :-1].T.reshape(-1)
    hs_top = _decoder(
        tgt_ids, tgt_emb, enc_bsh, W_a.astype(bf),
        W_ih_d0[:, :H].T.astype(bf), W_ih_d0[:, H:].T.astype(bf),
        W_hh_d0.T.astype(bf), b_d0.reshape(1, -1),
        jnp.stack([W_ih_d12[0].T, W_ih_d12[1].T]).astype(bf),
        jnp.stack([W_hh_d12[0].T, W_hh_d12[1].T]).astype(bf),
        b_d12, h_init, c_init, B)

    rows = hs_top.transpose(1, 0, 2).reshape(B * (T - 1), H)
    logits = _projection(rows, W_out.astype(bf), b_out.reshape(1, VT))
    return logits.reshape(B, T - 1, VT)


# speedup vs baseline: 1.0293x; 1.0031x over previous
import jax
import jax.numpy as jnp
from jax.experimental import pallas as pl
from jax.experimental.pallas import tpu as pltpu


def _dgT(x, wT):
    return jax.lax.dot_general(
        x.astype(jnp.bfloat16), wT, (((1,), (0,)), ((), ())),
        preferred_element_type=jnp.float32)


def _lstm_elem(g, c, fd):
    i = g[:, 0 * fd:1 * fd]
    f = g[:, 1 * fd:2 * fd]
    gg = g[:, 2 * fd:3 * fd]
    o = g[:, 3 * fd:4 * fd]
    c2 = jax.nn.sigmoid(f) * c + jax.nn.sigmoid(i) * jnp.tanh(gg)
    h2 = jax.nn.sigmoid(o) * jnp.tanh(c2)
    return h2, c2


def _gather_rows(ids_ref, table_ref, dst_ref, sem, n):
    def issue(i, carry):
        pltpu.make_async_copy(
            table_ref.at[ids_ref[i]], dst_ref.at[i], sem).start()
        return carry

    jax.lax.fori_loop(0, n, issue, 0)
    pltpu.make_async_copy(
        table_ref.at[pl.ds(0, n)], dst_ref.at[pl.ds(0, n)], sem).wait()


def _bilstm_core(xflat_ref, wih_ref, whh_ref, b_ref, out_ref, hf_ref, cf_ref,
                 gx_ref, S, B):
    Dh = whh_ref.shape[1]
    FD = 4 * Dh
    gx_ref[...] = (jax.lax.dot_general(
        xflat_ref[...].astype(jnp.bfloat16), wih_ref[...],
        (((1,), (0,)), ((), ())), preferred_element_type=jnp.float32)
        + b_ref[...]).astype(gx_ref.dtype)

    def step(t, carry):
        hF, cF, hB, cB = carry
        s = S - 1 - t
        gf = gx_ref[pl.ds(t * B, B), :FD] + _dgT(hF, whh_ref[0])
        gb = gx_ref[pl.ds(s * B, B), FD:] + _dgT(hB, whh_ref[1])
        hF2, cF2 = _lstm_elem(gf, cF, Dh)
        hB2, cB2 = _lstm_elem(gb, cB, Dh)
        out_ref[pl.ds(t * B, B), :Dh] = hF2.astype(out_ref.dtype)
        out_ref[pl.ds(s * B, B), Dh:] = hB2.astype(out_ref.dtype)
        return (hF2, cF2, hB2, cB2)

    z = jnp.zeros((B, Dh), jnp.float32)
    hF, cF, hB, cB = jax.lax.fori_loop(0, S, step, (z, z, z, z))
    hf_ref[0] = hF
    cf_ref[0] = cF
    hf_ref[1] = hB
    cf_ref[1] = cB


def _encoder_body(ids_ref, emb_ref, wih0_ref, whh0_ref, b0_ref,
                  wih1_ref, whh1_ref, b1_ref, wih2_ref, whh2_ref, b2_ref,
                  out_ref, hf_ref, cf_ref, seq0_ref, seqA_ref, seqB_ref,
                  gx_ref, sem):
    NTOK = seq0_ref.shape[0]
    B = hf_ref.shape[2]
    S = NTOK // B
    _gather_rows(ids_ref, emb_ref, seq0_ref, sem, NTOK)
    _bilstm_core(seq0_ref, wih0_ref, whh0_ref, b0_ref, seqA_ref,
                 hf_ref.at[0], cf_ref.at[0], gx_ref, S, B)
    _bilstm_core(seqA_ref, wih1_ref, whh1_ref, b1_ref, seqB_ref,
                 hf_ref.at[1], cf_ref.at[1], gx_ref, S, B)
    _bilstm_core(seqB_ref, wih2_ref, whh2_ref, b2_ref, out_ref,
                 hf_ref.at[2], cf_ref.at[2], gx_ref, S, B)


def _encoder(ids, src_emb, w0, w1, w2, B):
    NTOK = ids.shape[0]
    D = src_emb.shape[1]
    Dh = w0[1].shape[1]
    return pl.pallas_call(
        _encoder_body,
        out_shape=(
            jax.ShapeDtypeStruct((NTOK, 2 * Dh), jnp.float32),
            jax.ShapeDtypeStruct((3, 2, B, Dh), jnp.float32),
            jax.ShapeDtypeStruct((3, 2, B, Dh), jnp.float32),
        ),
        in_specs=[
            pl.BlockSpec(memory_space=pltpu.SMEM),
            pl.BlockSpec(memory_space=pl.ANY),
        ] + [pl.BlockSpec(memory_space=pltpu.VMEM)] * 9,
        scratch_shapes=[
            pltpu.VMEM((NTOK, D), jnp.float32),
            pltpu.VMEM((NTOK, 2 * Dh), jnp.bfloat16),
            pltpu.VMEM((NTOK, 2 * Dh), jnp.bfloat16),
            pltpu.VMEM((NTOK, 8 * Dh), jnp.bfloat16),
            pltpu.SemaphoreType.DMA,
        ],
        compiler_params=pltpu.CompilerParams(
            vmem_limit_bytes=55 * 1024 * 1024),
        name="encoder3_bilstm",
    )(ids, src_emb, w0[0], w0[1], w0[2], w1[0], w1[1], w1[2],
      w2[0], w2[1], w2[2])


def _dec_body(ids_ref, temb_ref, enc_ref, wat_ref, wihe_ref, wihc_ref,
              whh0_ref, b0_ref, wih12_ref, whh12_ref, b12_ref,
              h_init_ref, c_init_ref, hs_ref, emb_ref, gx0_ref, enc2_ref,
              sem):
    T, B, H = hs_ref.shape
    NTOK = emb_ref.shape[0]
    S = enc_ref.shape[0] // B

    _gather_rows(ids_ref, temb_ref, emb_ref, sem, NTOK)
    gx0_ref[...] = (jax.lax.dot_general(
        emb_ref[...].astype(jnp.bfloat16), wihe_ref[...],
        (((1,), (0,)), ((), ())), preferred_element_type=jnp.float32)
        + b0_ref[...])
    enc2_ref[...] = jax.lax.dot_general(
        enc_ref[...], wat_ref[...], (((1,), (0,)), ((), ())),
        preferred_element_type=jnp.float32).astype(jnp.bfloat16)

    def step(t, carry):
        h0, c0, h1, c1, h2, c2 = carry
        enc2 = enc2_ref[...].reshape(B, S, H)
        enc = enc_ref[...].reshape(B, S, H)
        scores = jnp.sum(h2.astype(jnp.bfloat16)[:, None, :] * enc2,
                         axis=2, dtype=jnp.float32)
        m = jnp.max(scores, axis=1, keepdims=True)
        e = jnp.exp(scores - m)
        attn = e / jnp.sum(e, axis=1, keepdims=True)
        ctx = jnp.sum(attn.astype(jnp.bfloat16)[:, :, None] * enc,
                      axis=1, dtype=jnp.float32)

        g0 = (gx0_ref[pl.ds(t * B, B), :] + _dgT(ctx, wihc_ref[...])
              + _dgT(h0, whh0_ref[...]))
        h0n, c0n = _lstm_elem(g0, c0, H)
        g1 = (_dgT(h0n, wih12_ref[0]) + _dgT(h1, whh12_ref[0])
              + b12_ref[0][None, :])
        h1n, c1n = _lstm_elem(g1, c1, H)
        g2 = (_dgT(h1n, wih12_ref[1]) + _dgT(h2, whh12_ref[1])
              + b12_ref[1][None, :])
        h2n, c2n = _lstm_elem(g2, c2, H)
        hs_ref[t] = h2n
        return (h0n, c0n, h1n, c1n, h2n, c2n)

    init = (h_init_ref[0], c_init_ref[0], h_init_ref[1], c_init_ref[1],
            h_init_ref[2], c_init_ref[2])
    jax.lax.fori_loop(0, T, step, init)


def _decoder(ids, tgt_emb, enc_flat, wat, wihe, wihc, whh0, b0, wih12,
             whh12, b12, h_init, c_init, B):
    NTOK = ids.shape[0]
    H = tgt_emb.shape[1]
    T = NTOK // B
    return pl.pallas_call(
        _dec_body,
        out_shape=jax.ShapeDtypeStruct((T, B, H), jnp.float32),
        in_specs=[
            pl.BlockSpec(memory_space=pltpu.SMEM),
            pl.BlockSpec(memory_space=pl.ANY),
        ] + [pl.BlockSpec(memory_space=pltpu.VMEM)] * 11,
        scratch_shapes=[
            pltpu.VMEM((NTOK, H), jnp.float32),
            pltpu.VMEM((NTOK, 4 * H), jnp.float32),
            pltpu.VMEM(enc_flat.shape, jnp.bfloat16),
            pltpu.SemaphoreType.DMA,
        ],
        compiler_params=pltpu.CompilerParams(
            vmem_limit_bytes=57 * 1024 * 1024 + 512 * 1024),
        name="decoder_recurrence",
    )(ids, tgt_emb, enc_flat, wat, wihe, wihc, whh0, b0, wih12, whh12, b12,
      h_init, c_init)


def _proj_body(x_ref, w_ref, b_ref, o_ref):
    o_ref[...] = (jax.lax.dot_general(
        x_ref[...].astype(jnp.bfloat16), w_ref[...],
        (((1,), (1,)), ((), ())), preferred_element_type=jnp.float32)
        + b_ref[...])


def _projection(x, w, b):
    R, H = x.shape
    V = w.shape[0]
    BV = 1280
    return pl.pallas_call(
        _proj_body,
        out_shape=jax.ShapeDtypeStruct((R, V), jnp.float32),
        grid=(V // BV,),
        in_specs=[
            pl.BlockSpec((R, H), lambda v: (0, 0)),
            pl.BlockSpec((BV, H), lambda v: (v, 0)),
            pl.BlockSpec((1, BV), lambda v: (0, v)),
        ],
        out_specs=pl.BlockSpec((R, BV), lambda v: (0, v)),
        compiler_params=pltpu.CompilerParams(
            dimension_semantics=("parallel",),
            vmem_limit_bytes=55 * 1024 * 1024),
        name="out_projection",
    )(x, w, b)


def kernel(x, y, src_emb, tgt_emb, W_ih_e0, W_hh_e0, b_e0, W_ih_e12,
           W_hh_e12, b_e12, W_ih_d0, W_hh_d0, b_d0, W_ih_d12, W_hh_d12,
           b_d12, W_a, W_out, b_out):
    B, S = x.shape
    T = y.shape[1]
    H = tgt_emb.shape[1]
    VT = W_out.shape[0]
    bf = jnp.bfloat16

    def enc_weights(wih, whh, b):
        wihT = jnp.concatenate([wih[0].T, wih[1].T], axis=1).astype(bf)
        whhT = jnp.stack([whh[0].T, whh[1].T]).astype(bf)
        bcat = jnp.concatenate([b[0], b[1]]).reshape(1, -1)
        return wihT, whhT, bcat

    src_ids = x.T.reshape(-1)
    seq, hf, cf = _encoder(
        src_ids, src_emb,
        enc_weights(W_ih_e0, W_hh_e0, b_e0),
        enc_weights(W_ih_e12[0], W_hh_e12[0], b_e12[0]),
        enc_weights(W_ih_e12[1], W_hh_e12[1], b_e12[1]), B)

    enc_flat = (seq.reshape(S, B, H).transpose(1, 0, 2)
                .reshape(S * B, H).astype(bf))
    h_init = jnp.stack(
        [jnp.concatenate([hf[l, 0], hf[l, 1]], -1) for l in range(3)])
    c_init = jnp.stack(
        [jnp.concatenate([cf[l, 0], cf[l, 1]], -1) for l in range(3)])

    tgt_ids = y[:, :-1].T.reshape(-1)
    hs_top = _decoder(
        tgt_ids, tgt_emb, enc_flat, W_a.T.astype(bf),
        W_ih_d0[:, :H].T.astype(bf), W_ih_d0[:, H:].T.astype(bf),
        W_hh_d0.T.astype(bf), b_d0.reshape(1, -1),
        jnp.stack([W_ih_d12[0].T, W_ih_d12[1].T]).astype(bf),
        jnp.stack([W_hh_d12[0].T, W_hh_d12[1].T]).astype(bf),
        b_d12, h_init, c_init, B)

    rows = hs_top.transpose(1, 0, 2).reshape(B * (T - 1), H)
    logits = _projection(rows, W_out.astype(bf), b_out.reshape(1, VT))
    return logits.reshape(B, T - 1, VT)


# speedup vs baseline: 1.0855x; 1.0546x over previous
import jax
import jax.numpy as jnp
from jax.experimental import pallas as pl
from jax.experimental.pallas import tpu as pltpu


def _dgT(x, wT):
    return jax.lax.dot_general(
        x.astype(jnp.bfloat16), wT, (((1,), (0,)), ((), ())),
        preferred_element_type=jnp.float32)


def _lstm_elem(g, c, fd):
    i = g[:, 0 * fd:1 * fd]
    f = g[:, 1 * fd:2 * fd]
    gg = g[:, 2 * fd:3 * fd]
    o = g[:, 3 * fd:4 * fd]
    c2 = jax.nn.sigmoid(f) * c + jax.nn.sigmoid(i) * jnp.tanh(gg)
    h2 = jax.nn.sigmoid(o) * jnp.tanh(c2)
    return h2, c2


def _gather_rows(ids_ref, table_ref, dst_ref, sem, n):
    def issue(i, carry):
        pltpu.make_async_copy(
            table_ref.at[ids_ref[i]], dst_ref.at[i], sem).start()
        return carry

    jax.lax.fori_loop(0, n, issue, 0)
    pltpu.make_async_copy(
        table_ref.at[pl.ds(0, n)], dst_ref.at[pl.ds(0, n)], sem).wait()


def _bilstm_core(xflat_ref, wih_ref, whh_ref, b_ref, out_ref, hf_ref, cf_ref,
                 gx_ref, S, B):
    Dh = whh_ref.shape[1]
    FD = 4 * Dh
    gx_ref[...] = (jax.lax.dot_general(
        xflat_ref[...].astype(jnp.bfloat16), wih_ref[...],
        (((1,), (0,)), ((), ())), preferred_element_type=jnp.float32)
        + b_ref[...]).astype(gx_ref.dtype)

    def step(t, carry):
        hF, cF, hB, cB = carry
        s = S - 1 - t
        gf = gx_ref[pl.ds(t * B, B), :FD] + _dgT(hF, whh_ref[0])
        gb = gx_ref[pl.ds(s * B, B), FD:] + _dgT(hB, whh_ref[1])
        hF2, cF2 = _lstm_elem(gf, cF, Dh)
        hB2, cB2 = _lstm_elem(gb, cB, Dh)
        out_ref[pl.ds(t * B, B), :Dh] = hF2.astype(out_ref.dtype)
        out_ref[pl.ds(s * B, B), Dh:] = hB2.astype(out_ref.dtype)
        return (hF2, cF2, hB2, cB2)

    z = jnp.zeros((B, Dh), jnp.float32)
    hF, cF, hB, cB = jax.lax.fori_loop(0, S, step, (z, z, z, z))
    hf_ref[0] = hF
    cf_ref[0] = cF
    hf_ref[1] = hB
    cf_ref[1] = cB


def _encoder_body(ids_ref, emb_ref, wih0_ref, whh0_ref, b0_ref,
                  wih1_ref, whh1_ref, b1_ref, wih2_ref, whh2_ref, b2_ref,
                  out_ref, hf_ref, cf_ref, seq0_ref, seqA_ref, seqB_ref,
                  gx_ref, sem):
    NTOK = seq0_ref.shape[0]
    B = hf_ref.shape[2]
    S = NTOK // B
    _gather_rows(ids_ref, emb_ref, seq0_ref, sem, NTOK)
    _bilstm_core(seq0_ref, wih0_ref, whh0_ref, b0_ref, seqA_ref,
                 hf_ref.at[0], cf_ref.at[0], gx_ref, S, B)
    _bilstm_core(seqA_ref, wih1_ref, whh1_ref, b1_ref, seqB_ref,
                 hf_ref.at[1], cf_ref.at[1], gx_ref, S, B)
    _bilstm_core(seqB_ref, wih2_ref, whh2_ref, b2_ref, out_ref,
                 hf_ref.at[2], cf_ref.at[2], gx_ref, S, B)


def _encoder(ids, src_emb, w0, w1, w2, B):
    NTOK = ids.shape[0]
    D = src_emb.shape[1]
    Dh = w0[1].shape[1]
    return pl.pallas_call(
        _encoder_body,
        out_shape=(
            jax.ShapeDtypeStruct((NTOK, 2 * Dh), jnp.float32),
            jax.ShapeDtypeStruct((3, 2, B, Dh), jnp.float32),
            jax.ShapeDtypeStruct((3, 2, B, Dh), jnp.float32),
        ),
        in_specs=[
            pl.BlockSpec(memory_space=pltpu.SMEM),
            pl.BlockSpec(memory_space=pl.ANY),
        ] + [pl.BlockSpec(memory_space=pltpu.VMEM)] * 9,
        scratch_shapes=[
            pltpu.VMEM((NTOK, D), jnp.float32),
            pltpu.VMEM((NTOK, 2 * Dh), jnp.bfloat16),
            pltpu.VMEM((NTOK, 2 * Dh), jnp.bfloat16),
            pltpu.VMEM((NTOK, 8 * Dh), jnp.bfloat16),
            pltpu.SemaphoreType.DMA,
        ],
        compiler_params=pltpu.CompilerParams(
            vmem_limit_bytes=55 * 1024 * 1024),
        name="encoder3_bilstm",
    )(ids, src_emb, w0[0], w0[1], w0[2], w1[0], w1[1], w1[2],
      w2[0], w2[1], w2[2])


def _dec_body(ids_ref, temb_ref, enc_ref, wat_ref, wihe_ref, wihc_ref,
              whh0_ref, b0_ref, wih12_ref, whh12_ref, b12_ref,
              h_init_ref, c_init_ref, hs_ref, emb_ref, gx0_ref, enc2_ref,
              sem):
    T, B, H = hs_ref.shape
    NTOK = emb_ref.shape[0]
    S = enc_ref.shape[0] // B

    _gather_rows(ids_ref, temb_ref, emb_ref, sem, NTOK)
    gx0_ref[...] = (jax.lax.dot_general(
        emb_ref[...].astype(jnp.bfloat16), wihe_ref[...],
        (((1,), (0,)), ((), ())), preferred_element_type=jnp.float32)
        + b0_ref[...])
    enc2_ref[...] = jax.lax.dot_general(
        enc_ref[...], wat_ref[...], (((1,), (0,)), ((), ())),
        preferred_element_type=jnp.float32).astype(jnp.bfloat16)

    SC = 32

    def step(t, carry):
        h0, c0, h1, c1, h2, c2 = carry
        h2b = h2.astype(jnp.bfloat16)
        scores = jnp.concatenate([
            jnp.sum(h2b[None, :, :]
                    * enc2_ref[pl.ds(k * SC * B, SC * B), :]
                    .reshape(SC, B, H),
                    axis=2, dtype=jnp.float32)
            for k in range(S // SC)], axis=0)
        m = jnp.max(scores, axis=0, keepdims=True)
        e = jnp.exp(scores - m)
        attn = (e / jnp.sum(e, axis=0, keepdims=True)).astype(jnp.bfloat16)
        ctx = sum(
            jnp.sum(attn[k * SC:(k + 1) * SC, :, None]
                    * enc_ref[pl.ds(k * SC * B, SC * B), :]
                    .reshape(SC, B, H),
                    axis=0, dtype=jnp.float32)
            for k in range(S // SC))

        g0 = (gx0_ref[pl.ds(t * B, B), :] + _dgT(ctx, wihc_ref[...])
              + _dgT(h0, whh0_ref[...]))
        h0n, c0n = _lstm_elem(g0, c0, H)
        g1 = (_dgT(h0n, wih12_ref[0]) + _dgT(h1, whh12_ref[0])
              + b12_ref[0][None, :])
        h1n, c1n = _lstm_elem(g1, c1, H)
        g2 = (_dgT(h1n, wih12_ref[1]) + _dgT(h2, whh12_ref[1])
              + b12_ref[1][None, :])
        h2n, c2n = _lstm_elem(g2, c2, H)
        hs_ref[t] = h2n
        return (h0n, c0n, h1n, c1n, h2n, c2n)

    init = (h_init_ref[0], c_init_ref[0], h_init_ref[1], c_init_ref[1],
            h_init_ref[2], c_init_ref[2])
    jax.lax.fori_loop(0, T, step, init)


def _decoder(ids, tgt_emb, enc_flat, wat, wihe, wihc, whh0, b0, wih12,
             whh12, b12, h_init, c_init, B):
    NTOK = ids.shape[0]
    H = tgt_emb.shape[1]
    T = NTOK // B
    return pl.pallas_call(
        _dec_body,
        out_shape=jax.ShapeDtypeStruct((T, B, H), jnp.float32),
        in_specs=[
            pl.BlockSpec(memory_space=pltpu.SMEM),
            pl.BlockSpec(memory_space=pl.ANY),
        ] + [pl.BlockSpec(memory_space=pltpu.VMEM)] * 11,
        scratch_shapes=[
            pltpu.VMEM((NTOK, H), jnp.float32),
            pltpu.VMEM((NTOK, 4 * H), jnp.float32),
            pltpu.VMEM(enc_flat.shape, jnp.bfloat16),
            pltpu.SemaphoreType.DMA,
        ],
        compiler_params=pltpu.CompilerParams(
            vmem_limit_bytes=57 * 1024 * 1024 + 512 * 1024),
        name="decoder_recurrence",
    )(ids, tgt_emb, enc_flat, wat, wihe, wihc, whh0, b0, wih12, whh12, b12,
      h_init, c_init)


def _proj_body(x_ref, w_ref, b_ref, o_ref):
    o_ref[...] = (jax.lax.dot_general(
        x_ref[...].astype(jnp.bfloat16), w_ref[...],
        (((1,), (1,)), ((), ())), preferred_element_type=jnp.float32)
        + b_ref[...])


def _projection(x, w, b):
    R, H = x.shape
    V = w.shape[0]
    BV = 1280
    return pl.pallas_call(
        _proj_body,
        out_shape=jax.ShapeDtypeStruct((R, V), jnp.float32),
        grid=(V // BV,),
        in_specs=[
            pl.BlockSpec((R, H), lambda v: (0, 0)),
            pl.BlockSpec((BV, H), lambda v: (v, 0)),
            pl.BlockSpec((1, BV), lambda v: (0, v)),
        ],
        out_specs=pl.BlockSpec((R, BV), lambda v: (0, v)),
        compiler_params=pltpu.CompilerParams(
            dimension_semantics=("parallel",),
            vmem_limit_bytes=55 * 1024 * 1024),
        name="out_projection",
    )(x, w, b)


def kernel(x, y, src_emb, tgt_emb, W_ih_e0, W_hh_e0, b_e0, W_ih_e12,
           W_hh_e12, b_e12, W_ih_d0, W_hh_d0, b_d0, W_ih_d12, W_hh_d12,
           b_d12, W_a, W_out, b_out):
    B, S = x.shape
    T = y.shape[1]
    H = tgt_emb.shape[1]
    VT = W_out.shape[0]
    bf = jnp.bfloat16

    def enc_weights(wih, whh, b):
        wihT = jnp.concatenate([wih[0].T, wih[1].T], axis=1).astype(bf)
        whhT = jnp.stack([whh[0].T, whh[1].T]).astype(bf)
        bcat = jnp.concatenate([b[0], b[1]]).reshape(1, -1)
        return wihT, whhT, bcat

    src_ids = x.T.reshape(-1)
    seq, hf, cf = _encoder(
        src_ids, src_emb,
        enc_weights(W_ih_e0, W_hh_e0, b_e0),
        enc_weights(W_ih_e12[0], W_hh_e12[0], b_e12[0]),
        enc_weights(W_ih_e12[1], W_hh_e12[1], b_e12[1]), B)

    enc_flat = seq.astype(bf)
    h_init = jnp.stack(
        [jnp.concatenate([hf[l, 0], hf[l, 1]], -1) for l in range(3)])
    c_init = jnp.stack(
        [jnp.concatenate([cf[l, 0], cf[l, 1]], -1) for l in range(3)])

    tgt_ids = y[:, :-1].T.reshape(-1)
    hs_top = _decoder(
        tgt_ids, tgt_emb, enc_flat, W_a.T.astype(bf),
        W_ih_d0[:, :H].T.astype(bf), W_ih_d0[:, H:].T.astype(bf),
        W_hh_d0.T.astype(bf), b_d0.reshape(1, -1),
        jnp.stack([W_ih_d12[0].T, W_ih_d12[1].T]).astype(bf),
        jnp.stack([W_hh_d12[0].T, W_hh_d12[1].T]).astype(bf),
        b_d12, h_init, c_init, B)

    rows = hs_top.transpose(1, 0, 2).reshape(B * (T - 1), H)
    logits = _projection(rows, W_out.astype(bf), b_out.reshape(1, VT))
    return logits.reshape(B, T - 1, VT)
